# baseline (speedup 1.0000x reference)
import functools

import jax
import jax.numpy as jnp
from jax import lax
from jax.experimental import pallas as pl
from jax.experimental.pallas import tpu as pltpu
from jax.experimental.pallas import tpu_sc as plsc

_NC = 2
_NS = 16
_NW = _NC * _NS
_SPT = 4


def _sc_embed(xf, tabf, b, s):
    n = b * s
    mesh = plsc.VectorSubcoreMesh(core_axis_name="c", subcore_axis_name="s")

    @functools.partial(
        pl.kernel,
        mesh=mesh,
        compiler_params=pltpu.CompilerParams(needs_layout_passes=False),
        out_type=jax.ShapeDtypeStruct((n * 4,), jnp.float32),
        scratch_types=[
            pltpu.VMEM((32,), jnp.float32),
            pltpu.VMEM((128 * s // 4,), jnp.int32),
            pltpu.VMEM((512 * s,), jnp.float32),
            pltpu.SemaphoreType.DMA,
        ],
    )
    def k(x_hbm, tab_hbm, out_hbm, tab_v, x_v, out_v, sem):
        wid = lax.axis_index("s") * _NC + lax.axis_index("c")
        pltpu.sync_copy(tab_hbm, tab_v)
        iota = lax.iota(jnp.int32, 16)
        lvec = [(g * 16 + iota) * (s // 4) for g in range(8)]

        def strip(t, carry):
            blk = wid * _SPT + t
            xoff = pl.multiple_of(blk * (128 * s // 4), 8)
            pltpu.sync_copy(x_hbm.at[pl.ds(xoff, 128 * s // 4)], x_v)

            def fire(j, c2):
                pltpu.make_async_copy(
                    out_v.at[pl.ds(j * 512, 512)],
                    out_hbm.at[pl.ds(j * 4 * b + blk * 512, 512)],
                    sem,
                ).start()
                return c2

            for jc in range(0, s, 40):

                @plsc.parallel_loop(jc, jc + 40, unroll=2)
                def tile_j(j):
                    o0 = j * 512
                    jw = j >> 2
                    jb = (j & 3) * 8
                    for g in range(8):
                        w = plsc.load_gather(x_v, [lvec[g] + jw])
                        xq = ((w >> jb) & 7) << 2
                        for c in range(4):
                            vals = plsc.load_gather(tab_v, [xq | c])
                            out_v[pl.ds(o0 + c * 128 + g * 16, 16)] = vals

                lax.fori_loop(jc, jc + 40, fire, 0)
            pltpu.make_async_copy(
                x_hbm.at[pl.ds(0, 512 * s)], out_v, sem
            ).wait()
            return carry

        lax.fori_loop(0, _SPT, strip, 0)

    return k(xf, tabf)


def kernel(x, table):
    b, s = x.shape
    n = b * s
    xf = lax.bitcast_convert_type(
        x.astype(jnp.int8).reshape(n // 4, 4), jnp.int32)
    tabf = table.reshape(32).astype(jnp.float32)
    flat = _sc_embed(xf, tabf, b, s)
    a4 = flat.reshape(s, b // 128, 4, 128)
    return a4.transpose(1, 3, 0, 2).reshape(b, s, 4)


# speedup vs baseline: 4.3844x; 4.3844x over previous
import functools

import jax
import jax.numpy as jnp
from jax import lax
from jax.experimental import pallas as pl
from jax.experimental.pallas import tpu as pltpu
from jax.experimental.pallas import tpu_sc as plsc

_NC = 2
_NS = 16
_NW = _NC * _NS
_SPT = 4


def _sc_embed(xf, tabf, b, s):
    n = b * s
    mesh = plsc.VectorSubcoreMesh(core_axis_name="c", subcore_axis_name="s")

    @functools.partial(
        pl.kernel,
        mesh=mesh,
        compiler_params=pltpu.CompilerParams(needs_layout_passes=False),
        out_type=jax.ShapeDtypeStruct((n * 4,), jnp.float32),
        scratch_types=[
            pltpu.VMEM((32,), jnp.float32),
            pltpu.VMEM((128 * s,), jnp.int32),
            pltpu.VMEM((512 * s,), jnp.float32),
            pltpu.SemaphoreType.DMA,
        ],
    )
    def k(x_hbm, tab_hbm, out_hbm, tab_v, x_v, out_v, sem):
        wid = lax.axis_index("s") * _NC + lax.axis_index("c")
        pltpu.sync_copy(tab_hbm, tab_v)
        iota = lax.iota(jnp.int32, 16)
        lvec = [(g * 16 + iota) * s for g in range(8)]

        def strip(t, carry):
            blk = wid * _SPT + t
            pltpu.sync_copy(x_hbm.at[pl.ds(blk * 128 * s, 128 * s)], x_v)

            def fire(j, c2):
                pltpu.make_async_copy(
                    out_v.at[pl.ds(j * 512, 512)],
                    out_hbm.at[pl.ds(j * 4 * b + blk * 512, 512)],
                    sem,
                ).start()
                return c2

            for jc in range(0, s, 40):

                @plsc.parallel_loop(jc, jc + 40, unroll=2)
                def tile_j(j):
                    o0 = j * 512
                    for g in range(8):
                        xcol = plsc.load_gather(x_v, [lvec[g] + j])
                        xq = xcol << 2
                        for c in range(4):
                            vals = plsc.load_gather(tab_v, [xq | c])
                            out_v[pl.ds(o0 + c * 128 + g * 16, 16)] = vals

                lax.fori_loop(jc, jc + 40, fire, 0)
            pltpu.make_async_copy(
                x_hbm.at[pl.ds(0, 512 * s)], out_v, sem
            ).wait()
            return carry

        lax.fori_loop(0, _SPT, strip, 0)

    return k(xf, tabf)


def kernel(x, table):
    b, s = x.shape
    n = b * s
    xf = x.astype(jnp.int32).reshape(n)
    tabf = table.reshape(32).astype(jnp.float32)
    flat = _sc_embed(xf, tabf, b, s)
    a4 = flat.reshape(s, b // 128, 4, 128)
    return a4.transpose(1, 3, 0, 2).reshape(b, s, 4)
